# baseline (speedup 1.0000x reference)
import functools

import jax
import jax.numpy as jnp
from jax import lax
from jax.experimental import pallas as pl
from jax.experimental.pallas import tpu as pltpu
from jax.experimental.pallas import tpu_sc as plsc

_BATCH = 16384
_FIELDS = 26
_N = _BATCH * _FIELDS
_TBL = 1216
_NKEYS = 1000
_NC, _NS, _L = 2, 16, 16
_NW = _NC * _NS
_CHUNK = _N // _NW
_ROWS = _BATCH // _NW
_SUB = 4
_SRO = _ROWS // _SUB
_SCH = _CHUNK // _SUB

_mesh = plsc.VectorSubcoreMesh(core_axis_name="c", subcore_axis_name="s")


@functools.partial(
    pl.kernel,
    mesh=_mesh,
    compiler_params=pltpu.CompilerParams(
        needs_layout_passes=False, use_tc_tiling_on_sc=False),
    out_type=jax.ShapeDtypeStruct((_BATCH, _FIELDS), jnp.int32),
    scratch_types=[
        pltpu.VMEM((_TBL,), jnp.int32),
        pltpu.VMEM((_NKEYS,), jnp.int32),
        pltpu.VMEM((_NKEYS,), jnp.int32),
        pltpu.VMEM((2, _SRO, _FIELDS), jnp.int32),
        pltpu.VMEM((2, _SRO, _FIELDS), jnp.int32),
        pltpu.SemaphoreType.DMA((2,)),
        pltpu.SemaphoreType.DMA((2,)),
    ],
)
def _lookup(in_hbm, keys_hbm, vals_hbm, out_hbm,
            table_v, keys_v, vals_v, in_v, out_v, sem_in, sem_out):
    wid = lax.axis_index("s") * _NC + lax.axis_index("c")
    row0 = wid * _ROWS

    in_dma = [None, None]
    for s in range(2):
        in_dma[s] = pltpu.async_copy(
            in_hbm.at[pl.ds(row0 + s * _SRO, _SRO), :], in_v.at[s],
            sem_in.at[s])

    pltpu.sync_copy(keys_hbm, keys_v)
    pltpu.sync_copy(vals_hbm, vals_v)

    zero = jnp.zeros((_L,), jnp.int32)

    @plsc.parallel_loop(0, _TBL // _L, unroll=8)
    def _zero_table(i):
        table_v[pl.ds(i * _L, _L)] = zero

    @plsc.parallel_loop(0, _NKEYS // _L, unroll=4)
    def _fill_table(i):
        k = keys_v[pl.ds(i * _L, _L)]
        v = vals_v[pl.ds(i * _L, _L)]
        plsc.store_scatter(table_v, [k], v)

    if _NKEYS % _L:
        k = keys_v[pl.ds(_NKEYS - _L, _L)]
        v = vals_v[pl.ds(_NKEYS - _L, _L)]
        plsc.store_scatter(table_v, [k], v)

    lanes = lax.iota(jnp.int32, _L)
    out_dma = [None, None]
    for s in range(_SUB):
        b = s % 2
        in_dma[b].wait()
        if out_dma[b] is not None:
            out_dma[b].wait()
        src = in_v.at[b]
        dst = out_v.at[b]

        @plsc.parallel_loop(0, _SCH // _L, unroll=16)
        def _gather(i):
            e = i * _L + lanes
            row = e // _FIELDS
            col = e - row * _FIELDS
            x = plsc.load_gather(src, [row, col])
            looked = plsc.load_gather(table_v, [x])
            plsc.store_scatter(dst, [row, col], looked)

        out_dma[b] = pltpu.async_copy(
            dst, out_hbm.at[pl.ds(row0 + s * _SRO, _SRO), :], sem_out.at[b])
        if s + 2 < _SUB:
            in_dma[b] = pltpu.async_copy(
                in_hbm.at[pl.ds(row0 + (s + 2) * _SRO, _SRO), :],
                in_v.at[b], sem_in.at[b])

    out_dma[0].wait()
    out_dma[1].wait()


@jax.jit
def kernel(inputs, keys, vals):
    values = _lookup(inputs, keys, vals)
    mask = inputs != 0
    return values, mask


# speedup vs baseline: 1.4235x; 1.4235x over previous
import functools

import jax
import jax.numpy as jnp
from jax import lax
from jax.experimental import pallas as pl
from jax.experimental.pallas import tpu as pltpu
from jax.experimental.pallas import tpu_sc as plsc

_BATCH = 16384
_FIELDS = 26
_N = _BATCH * _FIELDS
_TBL = 1216
_NKEYS = 1000
_NC, _NS, _L = 2, 16, 16
_NW = _NC * _NS
_CHUNK = _N // _NW
_ROWS = _BATCH // _NW
_SUB = 4
_SRO = _ROWS // _SUB
_SCH = _CHUNK // _SUB

_mesh = plsc.VectorSubcoreMesh(core_axis_name="c", subcore_axis_name="s")


@functools.partial(
    pl.kernel,
    mesh=_mesh,
    compiler_params=pltpu.CompilerParams(needs_layout_passes=False),
    out_type=jax.ShapeDtypeStruct((_BATCH, _FIELDS), jnp.int32),
    scratch_types=[
        pltpu.VMEM((_TBL,), jnp.int32),
        pltpu.VMEM((_NKEYS,), jnp.int32),
        pltpu.VMEM((_NKEYS,), jnp.int32),
        pltpu.VMEM((2, _SRO, _FIELDS), jnp.int32),
        pltpu.VMEM((2, _SRO, _FIELDS), jnp.int32),
        pltpu.SemaphoreType.DMA((2,)),
        pltpu.SemaphoreType.DMA((2,)),
    ],
)
def _lookup(in_hbm, keys_hbm, vals_hbm, out_hbm,
            table_v, keys_v, vals_v, in_v, out_v, sem_in, sem_out):
    wid = lax.axis_index("s") * _NC + lax.axis_index("c")
    row0 = wid * _ROWS

    in_dma = [None, None]
    for s in range(2):
        in_dma[s] = pltpu.async_copy(
            in_hbm.at[pl.ds(row0 + s * _SRO, _SRO), :], in_v.at[s],
            sem_in.at[s])

    pltpu.sync_copy(keys_hbm, keys_v)
    pltpu.sync_copy(vals_hbm, vals_v)

    zero = jnp.zeros((_L,), jnp.int32)

    @plsc.parallel_loop(0, _TBL // _L, unroll=8)
    def _zero_table(i):
        table_v[pl.ds(i * _L, _L)] = zero

    @plsc.parallel_loop(0, _NKEYS // _L, unroll=4)
    def _fill_table(i):
        k = keys_v[pl.ds(i * _L, _L)]
        v = vals_v[pl.ds(i * _L, _L)]
        plsc.store_scatter(table_v, [k], v)

    if _NKEYS % _L:
        k = keys_v[pl.ds(_NKEYS - _L, _L)]
        v = vals_v[pl.ds(_NKEYS - _L, _L)]
        plsc.store_scatter(table_v, [k], v)

    lanes = lax.iota(jnp.int32, _L)
    out_dma = [None, None]
    for s in range(_SUB):
        b = s % 2
        in_dma[b].wait()
        if out_dma[b] is not None:
            out_dma[b].wait()
        src = in_v.at[b]
        dst = out_v.at[b]

        @plsc.parallel_loop(0, _SCH // _L, unroll=16)
        def _gather(i):
            e = i * _L + lanes
            row = e // _FIELDS
            col = e - row * _FIELDS
            x = plsc.load_gather(src, [row, col])
            looked = plsc.load_gather(table_v, [x])
            plsc.store_scatter(dst, [row, col], looked)

        out_dma[b] = pltpu.async_copy(
            dst, out_hbm.at[pl.ds(row0 + s * _SRO, _SRO), :], sem_out.at[b])
        if s + 2 < _SUB:
            in_dma[b] = pltpu.async_copy(
                in_hbm.at[pl.ds(row0 + (s + 2) * _SRO, _SRO), :],
                in_v.at[b], sem_in.at[b])

    out_dma[0].wait()
    out_dma[1].wait()


@jax.jit
def kernel(inputs, keys, vals):
    values = _lookup(inputs, keys, vals)
    mask = inputs != 0
    return values, mask
